# Initial kernel scaffold; baseline (speedup 1.0000x reference)
import jax
import jax.numpy as jnp
from jax.experimental import pallas as pl


def kernel(x, edge_index, edge_attr, W_spline, root_w, gcn_bias, res_W, res_b, tcn_W, tcn_b):
    raise NotImplementedError("write your pallas kernel here")


# speedup vs baseline: 9.8920x; 9.8920x over previous
import functools
import jax
import jax.numpy as jnp
from jax import lax
from jax.experimental import pallas as pl
from jax.experimental.pallas import tpu as pltpu
from jax.experimental.pallas import tpu_sc as plsc

KS = 3
DIM = 3
K = KS ** DIM


def _elu(v):
    return jnp.where(v > 0, v, jnp.exp(jnp.minimum(v, 0.0)) - 1.0)


def _sc_gather(F, src, E, D, EC=128):
    V = F.shape[0]
    info = plsc.get_sparse_core_info()
    NC, NS = info.num_cores, info.num_subcores
    NW = NC * NS
    total_chunks = E // EC
    cpw = -(-total_chunks // NW)
    mesh = plsc.VectorSubcoreMesh(core_axis_name="c", subcore_axis_name="s")

    @functools.partial(
        pl.kernel,
        mesh=mesh,
        out_type=jax.ShapeDtypeStruct((E, D), jnp.float32),
        scratch_types=[
            pltpu.VMEM((EC,), jnp.int32),
            pltpu.VMEM((EC, D), jnp.float32),
            pltpu.SemaphoreType.DMA,
        ],
        compiler_params=pltpu.CompilerParams(use_tc_tiling_on_sc=False),
    )
    def k1(F_hbm, src_hbm, xs_hbm, idx_v, rows_v, sem):
        wid = lax.axis_index("s") * NC + lax.axis_index("c")

        def body(j, _):
            chunk = j * NW + wid

            @pl.when(chunk < total_chunks)
            def _():
                off = chunk * EC
                pltpu.sync_copy(src_hbm.at[pl.ds(off, EC)], idx_v)
                pltpu.async_copy(F_hbm.at[idx_v], rows_v, sem).wait()
                pltpu.sync_copy(rows_v, xs_hbm.at[pl.ds(off, EC)])
            return 0

        lax.fori_loop(0, cpw, body, 0)

    return k1(F, src)


def _tc_messages(ea, XSr, Wf, E, NB, C, Cout, ET=200):
    KC = K * C

    def k2(ea_ref, xs_ref, wf_ref, m_ref):
        ea_b = ea_ref[:]
        pos = ea_b * (KS - 1.0)
        lo = jnp.clip(jnp.floor(pos), 0, KS - 2)
        frac = pos - lo
        lo = lo.astype(jnp.int32)
        kiota = lax.broadcasted_iota(jnp.int32, (ET, KC), 1) // C
        B216 = jnp.zeros((ET, KC), jnp.float32)
        for c0 in (0, 1):
            for c1 in (0, 1):
                for c2 in (0, 1):
                    kidx = ((lo[:, 0] + c0) * (KS * KS)
                            + (lo[:, 1] + c1) * KS + (lo[:, 2] + c2))
                    w0 = frac[:, 0] if c0 else 1.0 - frac[:, 0]
                    w1 = frac[:, 1] if c1 else 1.0 - frac[:, 1]
                    w2 = frac[:, 2] if c2 else 1.0 - frac[:, 2]
                    w = w0 * w1 * w2
                    B216 = B216 + jnp.where(kidx[:, None] == kiota,
                                            w[:, None], 0.0)
        Bexp = jnp.broadcast_to(B216[:, None, :], (ET, NB, KC))
        Bexp = Bexp.reshape(ET * NB, KC)
        XSb = xs_ref[:]
        XSt = jnp.concatenate([XSb] * K, axis=1)
        P = (Bexp * XSt).astype(jnp.bfloat16)
        m_ref[...] = jnp.dot(P, wf_ref[:],
                             preferred_element_type=jnp.float32)

    grid = E // ET
    return pl.pallas_call(
        k2,
        grid=(grid,),
        in_specs=[
            pl.BlockSpec((ET, DIM), lambda i: (i, 0)),
            pl.BlockSpec((ET * NB, C), lambda i: (i, 0)),
            pl.BlockSpec((KC, Cout), lambda i: (0, 0)),
        ],
        out_specs=pl.BlockSpec((ET * NB, Cout), lambda i: (i, 0)),
        out_shape=jax.ShapeDtypeStruct((E * NB, Cout), jnp.float32),
    )(ea, XSr, Wf)


def _sc_scatter(M, dst, zeros_hbm, V, DH, E, EC=128):
    info = plsc.get_sparse_core_info()
    NC, NS = info.num_cores, info.num_subcores
    total_chunks = E // EC
    cps = -(-total_chunks // NS)
    rows_per_s = V // NS
    mesh = plsc.VectorSubcoreMesh(core_axis_name="c", subcore_axis_name="s")

    @functools.partial(
        pl.kernel,
        mesh=mesh,
        out_type=jax.ShapeDtypeStruct((2, V, DH), jnp.float32),
        scratch_types=[
            pltpu.VMEM((EC,), jnp.int32),
            pltpu.VMEM((EC, DH), jnp.float32),
            pltpu.VMEM_SHARED((V, DH), jnp.float32),
            pltpu.SemaphoreType.DMA,
        ],
        compiler_params=pltpu.CompilerParams(use_tc_tiling_on_sc=False),
    )
    def k3(m_hbm, dst_hbm, z_hbm, out_hbm, idx_v, m_v, acc, sem):
        c = lax.axis_index("c")
        s = lax.axis_index("s")
        pltpu.sync_copy(
            z_hbm.at[pl.ds(s * rows_per_s, rows_per_s)],
            acc.at[pl.ds(s * rows_per_s, rows_per_s)])
        plsc.subcore_barrier()

        def body(j, _):
            chunk = j * NS + s

            @pl.when(chunk < total_chunks)
            def _():
                off = chunk * EC
                pltpu.sync_copy(dst_hbm.at[pl.ds(off, EC)], idx_v)
                pltpu.sync_copy(m_hbm.at[pl.ds(off, EC), c], m_v)
                pltpu.sync_copy(m_v, acc.at[idx_v], add=True)
            return 0

        lax.fori_loop(0, cps, body, 0)
        plsc.subcore_barrier()
        pltpu.sync_copy(
            acc.at[pl.ds(s * rows_per_s, rows_per_s)],
            out_hbm.at[c, pl.ds(s * rows_per_s, rows_per_s)])

    return k3(M, dst, zeros_hbm)


def _tc_epilogue(S, F, RWBD, gb_t, RESBD, rb_t, TCNB, tb_t,
                 N, V, C, T, Cout, Tout, VT=400):
    NB = N * T
    D = NB * C
    DM = NB * Cout
    DO = N * Tout * Cout
    half = DM // 2

    def k4(s_ref, f_ref, rw_ref, gb_ref, rwt_ref, rb_ref, tw_ref, tb_ref,
           out_ref):
        Sm = jnp.concatenate([s_ref[0], s_ref[1]], axis=1)
        Fb = f_ref[:]
        sp = Sm + jnp.dot(Fb, rw_ref[:],
                          preferred_element_type=jnp.float32) + gb_ref[:]
        h1 = _elu(sp)
        res = _elu(jnp.dot(Fb, rwt_ref[:],
                           preferred_element_type=jnp.float32) + rb_ref[:])
        h2 = _elu(h1 + res)
        y = _elu(jnp.dot(h2, tw_ref[:],
                         preferred_element_type=jnp.float32) + tb_ref[:])
        out_ref[...] = y

    grid = V // VT
    return pl.pallas_call(
        k4,
        grid=(grid,),
        in_specs=[
            pl.BlockSpec((2, VT, half), lambda i: (0, i, 0)),
            pl.BlockSpec((VT, D), lambda i: (i, 0)),
            pl.BlockSpec((D, DM), lambda i: (0, 0)),
            pl.BlockSpec((1, DM), lambda i: (0, 0)),
            pl.BlockSpec((D, DM), lambda i: (0, 0)),
            pl.BlockSpec((1, DM), lambda i: (0, 0)),
            pl.BlockSpec((DM, DO), lambda i: (0, 0)),
            pl.BlockSpec((1, DO), lambda i: (0, 0)),
        ],
        out_specs=pl.BlockSpec((VT, DO), lambda i: (i, 0)),
        out_shape=jax.ShapeDtypeStruct((V, DO), jnp.float32),
    )(S, F, RWBD, gb_t, RESBD, rb_t, TCNB, tb_t)


def kernel(x, edge_index, edge_attr, W_spline, root_w, gcn_bias, res_W,
           res_b, tcn_W, tcn_b):
    N, V, C, T = x.shape
    NB = N * T
    Cout = W_spline.shape[-1]
    Tout = tcn_W.shape[0]
    E = edge_index.shape[1] // N
    D = NB * C

    src = edge_index[0, :E]
    dst = edge_index[1, :E]
    ea = edge_attr[:E]
    F = jnp.transpose(x, (1, 3, 0, 2)).reshape(V, D)
    Wf = W_spline.reshape(K * C, Cout).astype(jnp.bfloat16)
    res_Wt = jnp.transpose(res_W)
    half = (NB // 2) * Cout
    zeros_acc = jnp.zeros((V, half), jnp.float32)

    eyeNB = jnp.eye(NB, dtype=jnp.float32)
    RWBD = jnp.kron(eyeNB, root_w)
    RESBD = jnp.kron(eyeNB, res_Wt)
    gb_t = jnp.tile(gcn_bias, NB).reshape(1, NB * Cout)
    rb_t = jnp.tile(res_b, NB).reshape(1, NB * Cout)
    eyeN = jnp.eye(N, dtype=jnp.float32)
    eyeC = jnp.eye(Cout, dtype=jnp.float32)
    TCNB = jnp.einsum('st,nm,cd->tncmsd', tcn_W, eyeN, eyeC)
    TCNB = TCNB.reshape(NB * Cout, N * Tout * Cout)
    tb_t = jnp.tile(jnp.repeat(tcn_b, Cout), N).reshape(1, N * Tout * Cout)

    XS = _sc_gather(F, src, E, D)
    M = _tc_messages(ea, XS.reshape(E * NB, C), Wf, E, NB, C, Cout)
    S = _sc_scatter(M.reshape(E, 2, half), dst, zeros_acc, V, half, E)
    y = _tc_epilogue(S, F, RWBD, gb_t, RESBD, rb_t, TCNB, tb_t,
                     N, V, C, T, Cout, Tout)
    return jnp.transpose(y.reshape(V, N, Tout, Cout), (1, 0, 3, 2))


# speedup vs baseline: 22.5881x; 2.2835x over previous
import functools
import jax
import jax.numpy as jnp
from jax import lax
from jax.experimental import pallas as pl
from jax.experimental.pallas import tpu as pltpu
from jax.experimental.pallas import tpu_sc as plsc

KS = 3
DIM = 3
K = KS ** DIM


def _elu(v):
    return jnp.where(v > 0, v, jnp.exp(jnp.minimum(v, 0.0)) - 1.0)


def _tc_wmix(ea, Wf27, E, C, Cout, ET=2000):
    CO = C * Cout

    def kw(ea_ref, wf_ref, wm_ref):
        ea_b = ea_ref[:]
        pos = ea_b * (KS - 1.0)
        lo = jnp.clip(jnp.floor(pos), 0, KS - 2)
        frac = pos - lo
        lo = lo.astype(jnp.int32)
        kiota = lax.broadcasted_iota(jnp.int32, (ET, K), 1)
        B = jnp.zeros((ET, K), jnp.float32)
        for c0 in (0, 1):
            for c1 in (0, 1):
                for c2 in (0, 1):
                    kidx = ((lo[:, 0] + c0) * (KS * KS)
                            + (lo[:, 1] + c1) * KS + (lo[:, 2] + c2))
                    w0 = frac[:, 0] if c0 else 1.0 - frac[:, 0]
                    w1 = frac[:, 1] if c1 else 1.0 - frac[:, 1]
                    w2 = frac[:, 2] if c2 else 1.0 - frac[:, 2]
                    w = w0 * w1 * w2
                    B = B + jnp.where(kidx[:, None] == kiota,
                                      w[:, None], 0.0)
        wm_ref[...] = jnp.dot(B, wf_ref[:],
                              preferred_element_type=jnp.float32)

    grid = E // ET
    return pl.pallas_call(
        kw,
        grid=(grid,),
        in_specs=[
            pl.BlockSpec((ET, DIM), lambda i: (i, 0)),
            pl.BlockSpec((K, CO), lambda i: (0, 0)),
        ],
        out_specs=pl.BlockSpec((ET, CO), lambda i: (i, 0)),
        out_shape=jax.ShapeDtypeStruct((E, CO), jnp.float32),
    )(ea, Wf27)


def _sc_fused(F2, src, dst, Wmix, zeros_hbm, V, E, NB, C, Cout, EC=64):
    info = plsc.get_sparse_core_info()
    NC, NS = info.num_cores, info.num_subcores
    HB = NB // 2
    DH = HB * Cout
    DX = HB * C
    CO = C * Cout
    total_chunks = E // EC
    cps = -(-total_chunks // NS)
    rows_per_s = V // NS
    G = EC // 16
    BT = 2
    mesh = plsc.VectorSubcoreMesh(core_axis_name="c", subcore_axis_name="s")

    @functools.partial(
        pl.kernel,
        mesh=mesh,
        out_type=jax.ShapeDtypeStruct((2, V, DH), jnp.float32),
        scratch_types=[
            pltpu.VMEM((EC,), jnp.int32),
            pltpu.VMEM((EC,), jnp.int32),
            pltpu.VMEM((EC, DX), jnp.float32),
            pltpu.VMEM((EC, CO), jnp.float32),
            pltpu.VMEM((EC, DH), jnp.float32),
            pltpu.VMEM_SHARED((V, DH), jnp.float32),
            pltpu.SemaphoreType.DMA,
        ],
        compiler_params=pltpu.CompilerParams(use_tc_tiling_on_sc=False,
                                             needs_layout_passes=False),
    )
    def ks(f_hbm, src_hbm, dst_hbm, wm_hbm, z_hbm, out_hbm,
           src_v, dst_v, xs_v, wm_v, m_v, acc, sem):
        c = lax.axis_index("c")
        s = lax.axis_index("s")
        pltpu.sync_copy(
            z_hbm.at[pl.ds(s * rows_per_s, rows_per_s)],
            acc.at[pl.ds(s * rows_per_s, rows_per_s)])
        plsc.subcore_barrier()
        iota16 = lax.broadcasted_iota(jnp.int32, (16,), 0)

        def gbody(g, _):
            erow = iota16 + g * 16
            for bt in range(HB // BT):
                accv = [[jnp.zeros((16,), jnp.float32)
                         for _ in range(Cout)] for _ in range(BT)]
                for cc in range(C):
                    xg = [plsc.load_gather(
                        xs_v, [erow,
                               jnp.full((16,), (bt * BT + b2) * C + cc,
                                        jnp.int32)])
                          for b2 in range(BT)]
                    for o in range(Cout):
                        wv = plsc.load_gather(
                            wm_v, [erow,
                                   jnp.full((16,), cc * Cout + o,
                                            jnp.int32)])
                        for b2 in range(BT):
                            accv[b2][o] = accv[b2][o] + xg[b2] * wv
                for b2 in range(BT):
                    for o in range(Cout):
                        plsc.store_scatter(
                            m_v,
                            [erow,
                             jnp.full((16,), (bt * BT + b2) * Cout + o,
                                      jnp.int32)],
                            accv[b2][o])
            return 0

        def body(j, _):
            chunk = j * NS + s

            @pl.when(chunk < total_chunks)
            def _():
                off = chunk * EC
                pltpu.sync_copy(src_hbm.at[pl.ds(off, EC)], src_v)
                pltpu.async_copy(f_hbm.at[c].at[src_v], xs_v, sem).wait()
                pltpu.sync_copy(wm_hbm.at[pl.ds(off, EC)], wm_v)
                pltpu.sync_copy(dst_hbm.at[pl.ds(off, EC)], dst_v)
                lax.fori_loop(0, G, gbody, 0)
                pltpu.sync_copy(m_v, acc.at[dst_v], add=True)
            return 0

        lax.fori_loop(0, cps, body, 0)
        plsc.subcore_barrier()
        pltpu.sync_copy(
            acc.at[pl.ds(s * rows_per_s, rows_per_s)],
            out_hbm.at[c, pl.ds(s * rows_per_s, rows_per_s)])

    return ks(F2, src, dst, Wmix, zeros_hbm)


def _tc_epilogue(S, F, RWBD, gb_t, RESBD, rb_t, TCNB, tb_t,
                 N, V, C, T, Cout, Tout, VT=400):
    NB = N * T
    D = NB * C
    DM = NB * Cout
    DO = N * Tout * Cout
    half = DM // 2

    def k4(s_ref, f_ref, rw_ref, gb_ref, rwt_ref, rb_ref, tw_ref, tb_ref,
           out_ref):
        Sm = jnp.concatenate([s_ref[0], s_ref[1]], axis=1)
        Fb = f_ref[:]
        sp = Sm + jnp.dot(Fb, rw_ref[:],
                          preferred_element_type=jnp.float32) + gb_ref[:]
        h1 = _elu(sp)
        res = _elu(jnp.dot(Fb, rwt_ref[:],
                           preferred_element_type=jnp.float32) + rb_ref[:])
        h2 = _elu(h1 + res)
        y = _elu(jnp.dot(h2, tw_ref[:],
                         preferred_element_type=jnp.float32) + tb_ref[:])
        out_ref[...] = y

    grid = V // VT
    return pl.pallas_call(
        k4,
        grid=(grid,),
        in_specs=[
            pl.BlockSpec((2, VT, half), lambda i: (0, i, 0)),
            pl.BlockSpec((VT, D), lambda i: (i, 0)),
            pl.BlockSpec((D, DM), lambda i: (0, 0)),
            pl.BlockSpec((1, DM), lambda i: (0, 0)),
            pl.BlockSpec((D, DM), lambda i: (0, 0)),
            pl.BlockSpec((1, DM), lambda i: (0, 0)),
            pl.BlockSpec((DM, DO), lambda i: (0, 0)),
            pl.BlockSpec((1, DO), lambda i: (0, 0)),
        ],
        out_specs=pl.BlockSpec((VT, DO), lambda i: (i, 0)),
        out_shape=jax.ShapeDtypeStruct((V, DO), jnp.float32),
    )(S, F, RWBD, gb_t, RESBD, rb_t, TCNB, tb_t)


def kernel(x, edge_index, edge_attr, W_spline, root_w, gcn_bias, res_W,
           res_b, tcn_W, tcn_b):
    N, V, C, T = x.shape
    NB = N * T
    Cout = W_spline.shape[-1]
    Tout = tcn_W.shape[0]
    E = edge_index.shape[1] // N
    D = NB * C

    src = edge_index[0, :E]
    dst = edge_index[1, :E]
    ea = edge_attr[:E]
    F = jnp.transpose(x, (1, 3, 0, 2)).reshape(V, D)
    F2 = jnp.transpose(F.reshape(V, 2, (NB // 2) * C), (1, 0, 2))
    Wf27 = W_spline.reshape(K, C * Cout)
    res_Wt = jnp.transpose(res_W)
    half = (NB // 2) * Cout
    zeros_acc = jnp.zeros((V, half), jnp.float32)

    eyeNB = jnp.eye(NB, dtype=jnp.float32)
    RWBD = jnp.kron(eyeNB, root_w)
    RESBD = jnp.kron(eyeNB, res_Wt)
    gb_t = jnp.tile(gcn_bias, NB).reshape(1, NB * Cout)
    rb_t = jnp.tile(res_b, NB).reshape(1, NB * Cout)
    eyeN = jnp.eye(N, dtype=jnp.float32)
    eyeC = jnp.eye(Cout, dtype=jnp.float32)
    TCNB = jnp.einsum('st,nm,cd->tncmsd', tcn_W, eyeN, eyeC)
    TCNB = TCNB.reshape(NB * Cout, N * Tout * Cout)
    tb_t = jnp.tile(jnp.repeat(tcn_b, Cout), N).reshape(1, N * Tout * Cout)

    Wmix = _tc_wmix(ea, Wf27, E, C, Cout)
    S = _sc_fused(F2, src, dst, Wmix, zeros_acc, V, E, NB, C, Cout)
    y = _tc_epilogue(S, F, RWBD, gb_t, RESBD, rb_t, TCNB, tb_t,
                     N, V, C, T, Cout, Tout)
    return jnp.transpose(y.reshape(V, N, Tout, Cout), (1, 0, 3, 2))


# speedup vs baseline: 23.6217x; 1.0458x over previous
import functools
import jax
import jax.numpy as jnp
from jax import lax
from jax.experimental import pallas as pl
from jax.experimental.pallas import tpu as pltpu
from jax.experimental.pallas import tpu_sc as plsc

KS = 3
DIM = 3
K = KS ** DIM


def _elu(v):
    return jnp.where(v > 0, v, jnp.exp(jnp.minimum(v, 0.0)) - 1.0)


def _tc_wmix(ea, Wf27, E, C, Cout, ET=2000):
    CO = C * Cout

    def kw(ea_ref, wf_ref, wm_ref):
        ea_b = ea_ref[:]
        pos = ea_b * (KS - 1.0)
        lo = jnp.clip(jnp.floor(pos), 0, KS - 2)
        frac = pos - lo
        lo = lo.astype(jnp.int32)
        kiota = lax.broadcasted_iota(jnp.int32, (ET, K), 1)
        B = jnp.zeros((ET, K), jnp.float32)
        for c0 in (0, 1):
            for c1 in (0, 1):
                for c2 in (0, 1):
                    kidx = ((lo[:, 0] + c0) * (KS * KS)
                            + (lo[:, 1] + c1) * KS + (lo[:, 2] + c2))
                    w0 = frac[:, 0] if c0 else 1.0 - frac[:, 0]
                    w1 = frac[:, 1] if c1 else 1.0 - frac[:, 1]
                    w2 = frac[:, 2] if c2 else 1.0 - frac[:, 2]
                    w = w0 * w1 * w2
                    B = B + jnp.where(kidx[:, None] == kiota,
                                      w[:, None], 0.0)
        wm_ref[...] = jnp.dot(B, wf_ref[:],
                              preferred_element_type=jnp.float32)

    grid = E // ET
    return pl.pallas_call(
        kw,
        grid=(grid,),
        in_specs=[
            pl.BlockSpec((ET, DIM), lambda i: (i, 0)),
            pl.BlockSpec((K, CO), lambda i: (0, 0)),
        ],
        out_specs=pl.BlockSpec((ET, CO), lambda i: (i, 0)),
        out_shape=jax.ShapeDtypeStruct((E, CO), jnp.float32),
    )(ea, Wf27)


def _sc_fused(F2, src, dst, Wmix, zeros_hbm, V, E, NB, C, Cout, EC=64):
    info = plsc.get_sparse_core_info()
    NC, NS = info.num_cores, info.num_subcores
    HB = NB // 2
    DH = HB * Cout
    DX = HB * C
    CO = C * Cout
    total_chunks = E // EC
    cps = -(-total_chunks // NS)
    rows_per_s = V // NS
    G = EC // 16
    BT = 2
    mesh = plsc.VectorSubcoreMesh(core_axis_name="c", subcore_axis_name="s")

    @functools.partial(
        pl.kernel,
        mesh=mesh,
        out_type=jax.ShapeDtypeStruct((2, V, DH), jnp.float32),
        scratch_types=[
            pltpu.VMEM((EC,), jnp.int32),
            pltpu.VMEM((EC,), jnp.int32),
            pltpu.VMEM((EC, DX), jnp.float32),
            pltpu.VMEM((EC, CO), jnp.float32),
            pltpu.VMEM((EC, DH), jnp.float32),
            pltpu.VMEM_SHARED((V, DH), jnp.float32),
            pltpu.SemaphoreType.DMA,
            pltpu.SemaphoreType.DMA,
            pltpu.SemaphoreType.DMA,
            pltpu.SemaphoreType.DMA,
        ],
        compiler_params=pltpu.CompilerParams(use_tc_tiling_on_sc=False,
                                             needs_layout_passes=False),
    )
    def ks(f_hbm, src_hbm, dst_hbm, wm_hbm, z_hbm, out_hbm,
           src_v, dst_v, xs_v, wm_v, m_v, acc, sem, sem2, sem3, sem4):
        c = lax.axis_index("c")
        s = lax.axis_index("s")
        pltpu.sync_copy(
            z_hbm.at[pl.ds(s * rows_per_s, rows_per_s)],
            acc.at[pl.ds(s * rows_per_s, rows_per_s)])
        plsc.subcore_barrier()
        iota16 = lax.broadcasted_iota(jnp.int32, (16,), 0)

        def gbody(g, _):
            erow = iota16 + g * 16
            for bt in range(HB // BT):
                accv = [[jnp.zeros((16,), jnp.float32)
                         for _ in range(Cout)] for _ in range(BT)]
                for cc in range(C):
                    xg = [plsc.load_gather(
                        xs_v, [erow,
                               jnp.full((16,), (bt * BT + b2) * C + cc,
                                        jnp.int32)])
                          for b2 in range(BT)]
                    for o in range(Cout):
                        wv = plsc.load_gather(
                            wm_v, [erow,
                                   jnp.full((16,), cc * Cout + o,
                                            jnp.int32)])
                        for b2 in range(BT):
                            accv[b2][o] = accv[b2][o] + xg[b2] * wv
                for b2 in range(BT):
                    for o in range(Cout):
                        plsc.store_scatter(
                            m_v,
                            [erow,
                             jnp.full((16,), (bt * BT + b2) * Cout + o,
                                      jnp.int32)],
                            accv[b2][o])
            return 0

        def body(j, _):
            chunk = j * NS + s

            @pl.when(chunk < total_chunks)
            def _():
                off = chunk * EC
                h1 = pltpu.async_copy(src_hbm.at[pl.ds(off, EC)], src_v,
                                      sem)
                h2 = pltpu.async_copy(wm_hbm.at[pl.ds(off, EC)], wm_v,
                                      sem2)
                h3 = pltpu.async_copy(dst_hbm.at[pl.ds(off, EC)], dst_v,
                                      sem3)
                h1.wait()
                pltpu.async_copy(f_hbm.at[c].at[src_v], xs_v, sem4).wait()
                h2.wait()
                h3.wait()
                lax.fori_loop(0, G, gbody, 0)
                pltpu.sync_copy(m_v, acc.at[dst_v], add=True)
            return 0

        lax.fori_loop(0, cps, body, 0)
        plsc.subcore_barrier()
        pltpu.sync_copy(
            acc.at[pl.ds(s * rows_per_s, rows_per_s)],
            out_hbm.at[c, pl.ds(s * rows_per_s, rows_per_s)])

    return ks(F2, src, dst, Wmix, zeros_hbm)


def _tc_epilogue(S, F, RWBD, gb_t, RESBD, rb_t, TCNB, tb_t,
                 N, V, C, T, Cout, Tout, VT=400):
    NB = N * T
    D = NB * C
    DM = NB * Cout
    DO = N * Tout * Cout
    half = DM // 2

    def k4(s_ref, f_ref, rw_ref, gb_ref, rwt_ref, rb_ref, tw_ref, tb_ref,
           out_ref):
        Sm = jnp.concatenate([s_ref[0], s_ref[1]], axis=1)
        Fb = f_ref[:]
        sp = Sm + jnp.dot(Fb, rw_ref[:],
                          preferred_element_type=jnp.float32) + gb_ref[:]
        h1 = _elu(sp)
        res = _elu(jnp.dot(Fb, rwt_ref[:],
                           preferred_element_type=jnp.float32) + rb_ref[:])
        h2 = _elu(h1 + res)
        y = _elu(jnp.dot(h2, tw_ref[:],
                         preferred_element_type=jnp.float32) + tb_ref[:])
        out_ref[...] = y

    grid = V // VT
    return pl.pallas_call(
        k4,
        grid=(grid,),
        in_specs=[
            pl.BlockSpec((2, VT, half), lambda i: (0, i, 0)),
            pl.BlockSpec((VT, D), lambda i: (i, 0)),
            pl.BlockSpec((D, DM), lambda i: (0, 0)),
            pl.BlockSpec((1, DM), lambda i: (0, 0)),
            pl.BlockSpec((D, DM), lambda i: (0, 0)),
            pl.BlockSpec((1, DM), lambda i: (0, 0)),
            pl.BlockSpec((DM, DO), lambda i: (0, 0)),
            pl.BlockSpec((1, DO), lambda i: (0, 0)),
        ],
        out_specs=pl.BlockSpec((VT, DO), lambda i: (i, 0)),
        out_shape=jax.ShapeDtypeStruct((V, DO), jnp.float32),
    )(S, F, RWBD, gb_t, RESBD, rb_t, TCNB, tb_t)


def kernel(x, edge_index, edge_attr, W_spline, root_w, gcn_bias, res_W,
           res_b, tcn_W, tcn_b):
    N, V, C, T = x.shape
    NB = N * T
    Cout = W_spline.shape[-1]
    Tout = tcn_W.shape[0]
    E = edge_index.shape[1] // N
    D = NB * C

    src = edge_index[0, :E]
    dst = edge_index[1, :E]
    ea = edge_attr[:E]
    F = jnp.transpose(x, (1, 3, 0, 2)).reshape(V, D)
    F2 = jnp.transpose(F.reshape(V, 2, (NB // 2) * C), (1, 0, 2))
    Wf27 = W_spline.reshape(K, C * Cout)
    res_Wt = jnp.transpose(res_W)
    half = (NB // 2) * Cout
    zeros_acc = jnp.zeros((V, half), jnp.float32)

    eyeNB = jnp.eye(NB, dtype=jnp.float32)
    RWBD = jnp.kron(eyeNB, root_w)
    RESBD = jnp.kron(eyeNB, res_Wt)
    gb_t = jnp.tile(gcn_bias, NB).reshape(1, NB * Cout)
    rb_t = jnp.tile(res_b, NB).reshape(1, NB * Cout)
    eyeN = jnp.eye(N, dtype=jnp.float32)
    eyeC = jnp.eye(Cout, dtype=jnp.float32)
    TCNB = jnp.einsum('st,nm,cd->tncmsd', tcn_W, eyeN, eyeC)
    TCNB = TCNB.reshape(NB * Cout, N * Tout * Cout)
    tb_t = jnp.tile(jnp.repeat(tcn_b, Cout), N).reshape(1, N * Tout * Cout)

    Wmix = _tc_wmix(ea, Wf27, E, C, Cout)
    S = _sc_fused(F2, src, dst, Wmix, zeros_acc, V, E, NB, C, Cout)
    y = _tc_epilogue(S, F, RWBD, gb_t, RESBD, rb_t, TCNB, tb_t,
                     N, V, C, T, Cout, Tout)
    return jnp.transpose(y.reshape(V, N, Tout, Cout), (1, 0, 3, 2))


# speedup vs baseline: 48.0814x; 2.0355x over previous
import functools
import jax
import jax.numpy as jnp
from jax import lax
from jax.experimental import pallas as pl
from jax.experimental.pallas import tpu as pltpu
from jax.experimental.pallas import tpu_sc as plsc

KS = 3
DIM = 3
K = KS ** DIM


def _elu(v):
    return jnp.where(v > 0, v, jnp.exp(jnp.minimum(v, 0.0)) - 1.0)


def _tc_wmix(ea, Wf27, E, C, Cout, ET=3200):
    CO = C * Cout

    def kw(ea_ref, wf_ref, wm_ref):
        ea_b = ea_ref[:]
        pos = ea_b * (KS - 1.0)
        lo = jnp.clip(jnp.floor(pos), 0, KS - 2)
        frac = pos - lo
        lo = lo.astype(jnp.int32)
        kiota = lax.broadcasted_iota(jnp.int32, (ET, K), 1)
        B = jnp.zeros((ET, K), jnp.float32)
        for c0 in (0, 1):
            for c1 in (0, 1):
                for c2 in (0, 1):
                    kidx = ((lo[:, 0] + c0) * (KS * KS)
                            + (lo[:, 1] + c1) * KS + (lo[:, 2] + c2))
                    w0 = frac[:, 0] if c0 else 1.0 - frac[:, 0]
                    w1 = frac[:, 1] if c1 else 1.0 - frac[:, 1]
                    w2 = frac[:, 2] if c2 else 1.0 - frac[:, 2]
                    w = w0 * w1 * w2
                    B = B + jnp.where(kidx[:, None] == kiota,
                                      w[:, None], 0.0)
        wm_ref[...] = jnp.transpose(
            jnp.dot(B, wf_ref[:], preferred_element_type=jnp.float32))

    grid = E // ET
    return pl.pallas_call(
        kw,
        grid=(grid,),
        in_specs=[
            pl.BlockSpec((ET, DIM), lambda i: (i, 0)),
            pl.BlockSpec((K, CO), lambda i: (0, 0)),
        ],
        out_specs=pl.BlockSpec((CO, ET), lambda i: (0, i)),
        out_shape=jax.ShapeDtypeStruct((CO, E), jnp.float32),
    )(ea, Wf27)


def _sc_fused(F2, src, dst, Wmix, zeros_hbm, V, E, NB, C, Cout, EC=64):
    info = plsc.get_sparse_core_info()
    NC, NS = info.num_cores, info.num_subcores
    HB = NB // 2
    DH = HB * Cout
    DX = HB * C
    CO = C * Cout
    total_chunks = E // EC
    cps = -(-total_chunks // NS)
    rows_per_s = V // NS
    G = EC // 16
    BT = 2
    mesh = plsc.VectorSubcoreMesh(core_axis_name="c", subcore_axis_name="s")

    @functools.partial(
        pl.kernel,
        mesh=mesh,
        out_type=jax.ShapeDtypeStruct((2, V, DH), jnp.float32),
        scratch_types=[
            pltpu.VMEM((EC,), jnp.int32),
            pltpu.VMEM((EC,), jnp.int32),
            pltpu.VMEM((EC, DX), jnp.float32),
            pltpu.VMEM((CO, EC), jnp.float32),
            pltpu.VMEM((EC, DH), jnp.float32),
            pltpu.VMEM_SHARED((V, DH), jnp.float32),
            pltpu.SemaphoreType.DMA,
            pltpu.SemaphoreType.DMA,
            pltpu.SemaphoreType.DMA,
            pltpu.SemaphoreType.DMA,
        ],
        compiler_params=pltpu.CompilerParams(use_tc_tiling_on_sc=False,
                                             needs_layout_passes=False),
    )
    def ks(f_hbm, src_hbm, dst_hbm, wm_hbm, z_hbm, out_hbm,
           src_v, dst_v, xs_v, wm_v, m_v, acc, sem, sem2, sem3, sem4):
        c = lax.axis_index("c")
        s = lax.axis_index("s")
        pltpu.sync_copy(
            z_hbm.at[pl.ds(s * rows_per_s, rows_per_s)],
            acc.at[pl.ds(s * rows_per_s, rows_per_s)])
        plsc.subcore_barrier()
        iota16 = lax.broadcasted_iota(jnp.int32, (16,), 0)

        def gbody(g, _):
            erow = iota16 + g * 16
            for bt in range(HB // BT):
                accv = [[jnp.zeros((16,), jnp.float32)
                         for _ in range(Cout)] for _ in range(BT)]
                for cc in range(C):
                    xg = [plsc.load_gather(
                        xs_v, [erow,
                               jnp.full((16,), (bt * BT + b2) * C + cc,
                                        jnp.int32)])
                          for b2 in range(BT)]
                    for o in range(Cout):
                        wv = wm_v[cc * Cout + o, pl.ds(g * 16, 16)]
                        for b2 in range(BT):
                            accv[b2][o] = accv[b2][o] + xg[b2] * wv
                for b2 in range(BT):
                    for o in range(Cout):
                        plsc.store_scatter(
                            m_v,
                            [erow,
                             jnp.full((16,), (bt * BT + b2) * Cout + o,
                                      jnp.int32)],
                            accv[b2][o])
            return 0

        def body(j, _):
            chunk = j * NS + s

            @pl.when(chunk < total_chunks)
            def _():
                off = chunk * EC
                h1 = pltpu.async_copy(src_hbm.at[pl.ds(off, EC)], src_v,
                                      sem)
                h2 = pltpu.async_copy(wm_hbm.at[:, pl.ds(off, EC)],
                                      wm_v, sem2)
                h3 = pltpu.async_copy(dst_hbm.at[pl.ds(off, EC)], dst_v,
                                      sem3)
                h1.wait()
                pltpu.async_copy(f_hbm.at[c].at[src_v], xs_v, sem4).wait()
                h2.wait()
                h3.wait()
                lax.fori_loop(0, G, gbody, 0)
                pltpu.sync_copy(m_v, acc.at[dst_v], add=True)
            return 0

        lax.fori_loop(0, cps, body, 0)
        plsc.subcore_barrier()
        pltpu.sync_copy(
            acc.at[pl.ds(s * rows_per_s, rows_per_s)],
            out_hbm.at[c, pl.ds(s * rows_per_s, rows_per_s)])

    return ks(F2, src, dst, Wmix, zeros_hbm)


def _tc_epilogue(S, F, RWBD, gb_t, RESBD, rb_t, TCNB, tb_t,
                 N, V, C, T, Cout, Tout, VT=400):
    NB = N * T
    D = NB * C
    DM = NB * Cout
    DO = N * Tout * Cout
    half = DM // 2

    def k4(s_ref, f_ref, rw_ref, gb_ref, rwt_ref, rb_ref, tw_ref, tb_ref,
           out_ref):
        Sm = jnp.concatenate([s_ref[0], s_ref[1]], axis=1)
        Fb = f_ref[:]
        sp = Sm + jnp.dot(Fb, rw_ref[:],
                          preferred_element_type=jnp.float32) + gb_ref[:]
        h1 = _elu(sp)
        res = _elu(jnp.dot(Fb, rwt_ref[:],
                           preferred_element_type=jnp.float32) + rb_ref[:])
        h2 = _elu(h1 + res)
        y = _elu(jnp.dot(h2, tw_ref[:],
                         preferred_element_type=jnp.float32) + tb_ref[:])
        out_ref[...] = y

    grid = V // VT
    return pl.pallas_call(
        k4,
        grid=(grid,),
        in_specs=[
            pl.BlockSpec((2, VT, half), lambda i: (0, i, 0)),
            pl.BlockSpec((VT, D), lambda i: (i, 0)),
            pl.BlockSpec((D, DM), lambda i: (0, 0)),
            pl.BlockSpec((1, DM), lambda i: (0, 0)),
            pl.BlockSpec((D, DM), lambda i: (0, 0)),
            pl.BlockSpec((1, DM), lambda i: (0, 0)),
            pl.BlockSpec((DM, DO), lambda i: (0, 0)),
            pl.BlockSpec((1, DO), lambda i: (0, 0)),
        ],
        out_specs=pl.BlockSpec((VT, DO), lambda i: (i, 0)),
        out_shape=jax.ShapeDtypeStruct((V, DO), jnp.float32),
    )(S, F, RWBD, gb_t, RESBD, rb_t, TCNB, tb_t)


def kernel(x, edge_index, edge_attr, W_spline, root_w, gcn_bias, res_W,
           res_b, tcn_W, tcn_b):
    N, V, C, T = x.shape
    NB = N * T
    Cout = W_spline.shape[-1]
    Tout = tcn_W.shape[0]
    E = edge_index.shape[1] // N
    D = NB * C

    src = edge_index[0, :E]
    dst = edge_index[1, :E]
    ea = edge_attr[:E]
    F = jnp.transpose(x, (1, 3, 0, 2)).reshape(V, D)
    F2 = jnp.transpose(F.reshape(V, 2, (NB // 2) * C), (1, 0, 2))
    Wf27 = W_spline.reshape(K, C * Cout)
    res_Wt = jnp.transpose(res_W)
    half = (NB // 2) * Cout
    zeros_acc = jnp.zeros((V, half), jnp.float32)

    eyeNB = jnp.eye(NB, dtype=jnp.float32)
    RWBD = jnp.kron(eyeNB, root_w)
    RESBD = jnp.kron(eyeNB, res_Wt)
    gb_t = jnp.tile(gcn_bias, NB).reshape(1, NB * Cout)
    rb_t = jnp.tile(res_b, NB).reshape(1, NB * Cout)
    eyeN = jnp.eye(N, dtype=jnp.float32)
    eyeC = jnp.eye(Cout, dtype=jnp.float32)
    TCNB = jnp.einsum('st,nm,cd->tncmsd', tcn_W, eyeN, eyeC)
    TCNB = TCNB.reshape(NB * Cout, N * Tout * Cout)
    tb_t = jnp.tile(jnp.repeat(tcn_b, Cout), N).reshape(1, N * Tout * Cout)

    Wmix = _tc_wmix(ea, Wf27, E, C, Cout)
    S = _sc_fused(F2, src, dst, Wmix, zeros_acc, V, E, NB, C, Cout)
    y = _tc_epilogue(S, F, RWBD, gb_t, RESBD, rb_t, TCNB, tb_t,
                     N, V, C, T, Cout, Tout)
    return jnp.transpose(y.reshape(V, N, Tout, Cout), (1, 0, 3, 2))
